# baseline (speedup 1.0000x reference)
import functools

import jax
import jax.numpy as jnp
from jax import lax
from jax.experimental import pallas as pl
from jax.experimental.pallas import tpu as pltpu
from jax.experimental.pallas import tpu_sc as plsc

_N = 16777216
_NC = 2
_NS = 16
_L = 16
_NW = _NC * _NS
_PER_W = _N // _NW
_CHUNK = 16384
_NDMA = _PER_W // _CHUNK


@functools.partial(
    pl.kernel,
    mesh=plsc.VectorSubcoreMesh(core_axis_name="c", subcore_axis_name="s"),
    out_type=jax.ShapeDtypeStruct((_N,), jnp.float32),
    scratch_types=[
        pltpu.VMEM((_CHUNK,), jnp.float32),
        pltpu.VMEM((_L,), jnp.int32),
        pltpu.VMEM((_L,), jnp.float32),
        pltpu.VMEM((_L,), jnp.float32),
        pltpu.SemaphoreType.DMA,
        pltpu.SemaphoreType.DMA,
        pltpu.SemaphoreType.DMA,
    ],
)
def _sc_fill(idx_hbm, grad_hbm, out_hbm, zbuf, iv, gvec, gtile,
             sem, sem_s, sem_o):
    c = lax.axis_index("c")
    s = lax.axis_index("s")
    wid = c * _NS + s
    base = wid * _PER_W

    scalar_copies = [
        pltpu.make_async_copy(idx_hbm, iv.at[pl.ds(0, 1)], sem_s),
        pltpu.make_async_copy(grad_hbm, gvec.at[pl.ds(0, 1)], sem_s),
    ]
    for cp in scalar_copies:
        cp.start()

    zeros16 = jnp.zeros((_L,), jnp.float32)
    _UNROLL = 16

    def _zero_body(i, carry):
        for j in range(_UNROLL):
            zbuf[pl.ds((i * _UNROLL + j) * _L, _L)] = zeros16
        return carry

    lax.fori_loop(0, _CHUNK // (_L * _UNROLL), _zero_body, 0)

    for cp in scalar_copies:
        cp.wait()
    sidx = iv[...][0] % _N
    g0 = gvec[...][0]
    owner = sidx // _PER_W
    ochunk = (sidx % _PER_W) // _CHUNK
    is_owner = owner == wid

    copies = []
    ocopies = []
    for j in range(_NDMA):
        dst = out_hbm.at[pl.ds(base + j * _CHUNK, _CHUNK)]
        zc = pltpu.make_async_copy(zbuf, dst, sem)
        oc = pltpu.make_async_copy(zbuf, dst, sem_o)
        hit = is_owner & (ochunk == j)

        @pl.when(~hit)
        def _():
            zc.start()

        @pl.when(hit)
        def _():
            oc.start()

        copies.append(zc)
        ocopies.append(oc)

    @pl.when(is_owner)
    def _():
        ocopies[0].wait()
        aligned = jnp.minimum((sidx // 8) * 8, base + _PER_W - _L)
        off = sidx - aligned
        lanes = lax.iota(jnp.int32, _L)
        gtile[...] = jnp.where(lanes == off, g0, 0.0)
        pltpu.sync_copy(gtile, out_hbm.at[pl.ds(aligned, _L)])

    for j in range(_NDMA - 1):
        copies[j].wait()

    @pl.when(~is_owner)
    def _():
        copies[_NDMA - 1].wait()


def kernel(grad_output, input_sizes, dim, index, out):
    del input_sizes, dim, out
    idx1 = jnp.asarray(index, jnp.int32).reshape((1,))
    grad1 = jnp.asarray(grad_output, jnp.float32).reshape((1,))
    return _sc_fill(idx1, grad1)


# speedup vs baseline: 1.0193x; 1.0193x over previous
import functools

import jax
import jax.numpy as jnp
from jax import lax
from jax.experimental import pallas as pl
from jax.experimental.pallas import tpu as pltpu
from jax.experimental.pallas import tpu_sc as plsc

_N = 16777216
_NC = 2
_NS = 16
_L = 16
_NW = _NC * _NS
_PER_W = _N // _NW
_CHUNK = 16384
_NDMA = _PER_W // _CHUNK


@functools.partial(
    pl.kernel,
    mesh=plsc.VectorSubcoreMesh(core_axis_name="c", subcore_axis_name="s"),
    out_type=jax.ShapeDtypeStruct((_N,), jnp.float32),
    scratch_types=[
        pltpu.VMEM((_CHUNK,), jnp.float32),
        pltpu.VMEM((_L,), jnp.int32),
        pltpu.VMEM((_L,), jnp.float32),
        pltpu.VMEM((_L,), jnp.float32),
        pltpu.SemaphoreType.DMA,
        pltpu.SemaphoreType.DMA,
    ],
)
def _sc_fill(idx_hbm, grad_hbm, out_hbm, zbuf, iv, gvec, gtile, sem, sem_s):
    c = lax.axis_index("c")
    s = lax.axis_index("s")
    wid = c * _NS + s
    base = wid * _PER_W

    scalar_copies = [
        pltpu.make_async_copy(idx_hbm, iv.at[pl.ds(0, 1)], sem_s),
        pltpu.make_async_copy(grad_hbm, gvec.at[pl.ds(0, 1)], sem_s),
    ]
    for cp in scalar_copies:
        cp.start()

    zeros16 = jnp.zeros((_L,), jnp.float32)
    _UNROLL = 16

    def _zero_body(i, carry):
        for j in range(_UNROLL):
            zbuf[pl.ds((i * _UNROLL + j) * _L, _L)] = zeros16
        return carry

    lax.fori_loop(0, _CHUNK // (_L * _UNROLL), _zero_body, 0)

    def _fire(j, carry):
        off = pl.multiple_of(base + j * _CHUNK, 8)
        pltpu.make_async_copy(zbuf, out_hbm.at[pl.ds(off, _CHUNK)], sem).start()
        return carry

    lax.fori_loop(0, _NDMA, _fire, 0)

    for cp in scalar_copies:
        cp.wait()
    sidx = iv[...][0] % _N
    g0 = gvec[...][0]

    def _drain(j, carry):
        pltpu.make_async_copy(
            zbuf, out_hbm.at[pl.ds(base, _CHUNK)], sem).wait()
        return carry

    lax.fori_loop(0, _NDMA, _drain, 0)

    @pl.when(sidx // _PER_W == wid)
    def _():
        aligned = jnp.minimum((sidx // 8) * 8, base + _PER_W - _L)
        off = sidx - aligned
        lanes = lax.iota(jnp.int32, _L)
        gtile[...] = jnp.where(lanes == off, g0, 0.0)
        pltpu.sync_copy(gtile, out_hbm.at[pl.ds(aligned, _L)])


def kernel(grad_output, input_sizes, dim, index, out):
    del input_sizes, dim, out
    idx1 = jnp.asarray(index, jnp.int32).reshape((1,))
    grad1 = jnp.asarray(grad_output, jnp.float32).reshape((1,))
    return _sc_fill(idx1, grad1)


# speedup vs baseline: 1.5371x; 1.5080x over previous
import jax
import jax.numpy as jnp
from jax import lax
from jax.experimental import pallas as pl
from jax.experimental.pallas import tpu as pltpu

_N = 16777216
_CH = 524288
_NCOPIES = _N // _CH
_NSEM = 8


def _fill_body(idx_ref, grad_ref, out_ref, zbuf, gbuf, sems):
    target = idx_ref[0]
    kstar = target // _CH
    off = target % _CH

    zbuf[...] = jnp.zeros_like(zbuf)
    pos = lax.broadcasted_iota(jnp.int32, (_CH,), 0)
    gbuf[...] = jnp.where(pos == off, grad_ref[0], 0.0)

    copies = []
    for k in range(_NCOPIES):
        dst = out_ref.at[pl.ds(k * _CH, _CH)]
        sem = sems.at[k % _NSEM]
        zc = pltpu.make_async_copy(zbuf, dst, sem)
        gc = pltpu.make_async_copy(gbuf, dst, sem)

        @pl.when(kstar != k)
        def _():
            zc.start()

        @pl.when(kstar == k)
        def _():
            gc.start()

        copies.append(zc)
    for c in copies:
        c.wait()


def kernel(grad_output, input_sizes, dim, index, out):
    n = out.shape[0]
    idx = ((jnp.asarray(index, jnp.int32) + jnp.asarray(dim, jnp.int32))
           % jnp.asarray(input_sizes, jnp.int32)).reshape((1,))
    gval = jnp.asarray(grad_output, jnp.float32).reshape((1,))
    res = pl.pallas_call(
        _fill_body,
        in_specs=[pl.BlockSpec(memory_space=pltpu.SMEM),
                  pl.BlockSpec(memory_space=pltpu.SMEM)],
        out_specs=pl.BlockSpec(memory_space=pl.ANY),
        out_shape=jax.ShapeDtypeStruct((n,), jnp.float32),
        scratch_shapes=[
            pltpu.VMEM((_CH,), jnp.float32),
            pltpu.VMEM((_CH,), jnp.float32),
            pltpu.SemaphoreType.DMA((_NSEM,)),
        ],
    )(idx, gval)
    return res
